# baseline (device time: 110193 ns/iter reference)
import jax
import jax.numpy as jnp
from jax import lax
from jax.experimental import pallas as pl
from jax.experimental.pallas import tpu as pltpu

N_DEV = 8
SQ = 256
D = 1024
DH = 128
H_PER = 8
SCALE = 0.08838834764831843


def kernel(x, Wq, Wo, Wk, Wv):
    def body(x_ref, wq_ref, wo_ref, wk_ref, wv_ref, out_ref,
             comm_ref, send_sems, recv_sems):
        my = lax.axis_index("i")
        left = (my + N_DEV - 1) % N_DEV
        right = (my + 1) % N_DEV

        barrier = pltpu.get_barrier_semaphore()
        for nbr in (left, right):
            pl.semaphore_signal(barrier, inc=1, device_id=(nbr,),
                                device_id_type=pl.DeviceIdType.MESH)
        pl.semaphore_wait(barrier, 2)

        xv = x_ref[0]
        q = jnp.dot(xv, wq_ref[...], preferred_element_type=jnp.float32)
        k = jnp.dot(xv, wk_ref[...], preferred_element_type=jnp.float32)
        v = jnp.dot(xv, wv_ref[...], preferred_element_type=jnp.float32)

        outs = []
        for h in range(H_PER):
            sl = slice(h * DH, (h + 1) * DH)
            s = jnp.dot(q[:, sl], k[:, sl].T,
                        preferred_element_type=jnp.float32) * SCALE
            m = jnp.max(s, axis=-1, keepdims=True)
            p = jnp.exp(s - m)
            l = jnp.sum(p, axis=-1, keepdims=True)
            outs.append(jnp.dot(p / l, v[:, sl],
                                preferred_element_type=jnp.float32))
        attn = jnp.concatenate(outs, axis=1)
        partial = jnp.dot(attn, wo_ref[...],
                          preferred_element_type=jnp.float32)

        comm_ref[0] = partial
        out_ref[0] = partial

        for hop in range(N_DEV - 1):
            rdma = pltpu.make_async_remote_copy(
                src_ref=comm_ref.at[hop],
                dst_ref=comm_ref.at[hop + 1],
                send_sem=send_sems.at[hop],
                recv_sem=recv_sems.at[hop],
                device_id=(right,),
                device_id_type=pl.DeviceIdType.MESH,
            )
            rdma.start()
            rdma.wait()
            out_ref[0] += comm_ref[hop + 1]

    return pl.pallas_call(
        body,
        out_shape=jax.ShapeDtypeStruct((1, SQ, D), jnp.float32),
        in_specs=[pl.BlockSpec(memory_space=pltpu.VMEM)] * 5,
        out_specs=pl.BlockSpec(memory_space=pltpu.VMEM),
        scratch_shapes=[
            pltpu.VMEM((N_DEV, SQ, D), jnp.float32),
            pltpu.SemaphoreType.DMA((N_DEV - 1,)),
            pltpu.SemaphoreType.DMA((N_DEV - 1,)),
        ],
        compiler_params=pltpu.CompilerParams(collective_id=0),
    )(x, Wq, Wo, Wk, Wv)


# device time: 47704 ns/iter; 2.3099x vs baseline; 2.3099x over previous
import jax
import jax.numpy as jnp
from jax import lax
from jax.experimental import pallas as pl
from jax.experimental.pallas import tpu as pltpu

N_DEV = 8
SQ = 256
D = 1024
DH = 128
H_PER = 8
SCALE = 0.08838834764831843

MASKS = (1, 3, 4)


def kernel(x, Wq, Wo, Wk, Wv):
    def body(x_ref, wq_ref, wo_ref, wk_ref, wv_ref, out_ref,
             acc_ref, rs_recv, send_sems, recv_sems):
        my = lax.axis_index("i")
        b0 = my & 1
        b1 = (my >> 1) & 1
        b2 = (my >> 2) & 1
        h1 = b0 ^ b1
        h2 = b1
        h3 = b2
        partners = [my ^ m for m in MASKS]

        barrier = pltpu.get_barrier_semaphore()
        for p in partners:
            pl.semaphore_signal(barrier, inc=1, device_id=(p,),
                                device_id_type=pl.DeviceIdType.MESH)
        pl.semaphore_wait(barrier, 3)

        xv = x_ref[0]
        q = jnp.dot(xv, wq_ref[...], preferred_element_type=jnp.float32)
        k = jnp.dot(xv, wk_ref[...], preferred_element_type=jnp.float32)
        v = jnp.dot(xv, wv_ref[...], preferred_element_type=jnp.float32)

        outs = []
        for h in range(H_PER):
            sl = slice(h * DH, (h + 1) * DH)
            s = jnp.dot(q[:, sl], k[:, sl].T,
                        preferred_element_type=jnp.float32) * SCALE
            m = jnp.max(s, axis=-1, keepdims=True)
            p = jnp.exp(s - m)
            l = jnp.sum(p, axis=-1, keepdims=True)
            outs.append(jnp.dot(p / l, v[:, sl],
                                preferred_element_type=jnp.float32))
        attn = jnp.concatenate(outs, axis=1)
        acc_ref[...] = jnp.dot(attn, wo_ref[...],
                               preferred_element_type=jnp.float32)

        k0 = h1 * 128
        k1 = k0 + h2 * 64
        k2 = k1 + h3 * 32

        rs_plan = [
            ((1 - h1) * 128, 128, 0, k0),
            (k0 + (1 - h2) * 64, 64, 128, k1),
            (k1 + (1 - h3) * 32, 32, 192, k2),
        ]
        for s, (src_row, sz, stage_off, dst_row) in enumerate(rs_plan):
            rdma = pltpu.make_async_remote_copy(
                src_ref=acc_ref.at[pl.ds(src_row, sz), :],
                dst_ref=rs_recv.at[pl.ds(stage_off, sz), :],
                send_sem=send_sems.at[s],
                recv_sem=recv_sems.at[s],
                device_id=(partners[s],),
                device_id_type=pl.DeviceIdType.MESH,
            )
            rdma.start()
            rdma.wait()
            acc_ref[pl.ds(dst_row, sz), :] += rs_recv[pl.ds(stage_off, sz), :]

        ag_plan = [(2, k2, 32), (1, k1, 64), (0, k0, 128)]
        for s, (m_i, row, sz) in enumerate(ag_plan):
            rdma = pltpu.make_async_remote_copy(
                src_ref=acc_ref.at[pl.ds(row, sz), :],
                dst_ref=acc_ref.at[pl.ds(row, sz), :],
                send_sem=send_sems.at[3 + s],
                recv_sem=recv_sems.at[3 + s],
                device_id=(partners[m_i],),
                device_id_type=pl.DeviceIdType.MESH,
            )
            rdma.start()
            rdma.wait()

        out_ref[0] = acc_ref[...]

    return pl.pallas_call(
        body,
        out_shape=jax.ShapeDtypeStruct((1, SQ, D), jnp.float32),
        in_specs=[pl.BlockSpec(memory_space=pltpu.VMEM)] * 5,
        out_specs=pl.BlockSpec(memory_space=pltpu.VMEM),
        scratch_shapes=[
            pltpu.VMEM((SQ, D), jnp.float32),
            pltpu.VMEM((SQ, D), jnp.float32),
            pltpu.SemaphoreType.DMA((6,)),
            pltpu.SemaphoreType.DMA((6,)),
        ],
        compiler_params=pltpu.CompilerParams(collective_id=0),
    )(x, Wq, Wo, Wk, Wv)


# device time: 36501 ns/iter; 3.0189x vs baseline; 1.3069x over previous
import jax
import jax.numpy as jnp
from jax import lax
from jax.experimental import pallas as pl
from jax.experimental.pallas import tpu as pltpu

N_DEV = 8
SQ = 256
D = 1024
DH = 128
H_PER = 8
SCALE = 0.08838834764831843

RAIL_MASKS = ((1, 3, 4), (3, 4, 1), (4, 1, 3))
RAIL_COLS = ((0, 384), (384, 768), (768, 1024))
STAGE_OFF = (0, 128, 192)


def kernel(x, Wq, Wo, Wk, Wv):
    def body(x_ref, wq_ref, wo_ref, wk_ref, wv_ref, out_ref,
             acc_ref, rs_recv, send_sems, recv_sems):
        my = lax.axis_index("i")
        b0 = my & 1
        b1 = (my >> 1) & 1
        b2 = (my >> 2) & 1
        sels = (
            (b0 ^ b1, b1, b2),
            (b1, b2, b0),
            (b2, b0 ^ b1, b1),
        )

        barrier = pltpu.get_barrier_semaphore()
        for msk in (1, 3, 4):
            pl.semaphore_signal(barrier, inc=1, device_id=(my ^ msk,),
                                device_id_type=pl.DeviceIdType.MESH)
        pl.semaphore_wait(barrier, 3)

        xv = x_ref[0]
        q = jnp.dot(xv, wq_ref[...], preferred_element_type=jnp.float32)
        k = jnp.dot(xv, wk_ref[...], preferred_element_type=jnp.float32)
        v = jnp.dot(xv, wv_ref[...], preferred_element_type=jnp.float32)

        outs = []
        for h in range(H_PER):
            sl = slice(h * DH, (h + 1) * DH)
            s = jnp.dot(q[:, sl], k[:, sl].T,
                        preferred_element_type=jnp.float32) * SCALE
            m = jnp.max(s, axis=-1, keepdims=True)
            p = jnp.exp(s - m)
            l = jnp.sum(p, axis=-1, keepdims=True)
            outs.append(jnp.dot(p / l, v[:, sl],
                                preferred_element_type=jnp.float32))
        attn = jnp.concatenate(outs, axis=1)
        acc_ref[...] = jnp.dot(attn, wo_ref[...],
                               preferred_element_type=jnp.float32)

        plans = []
        for r, ((m0, m1, m2), (s0, s1, s2)) in enumerate(
                zip(RAIL_MASKS, sels)):
            k0 = s0 * 128
            k1 = k0 + s1 * 64
            k2 = k1 + s2 * 32
            c0, c1 = RAIL_COLS[r]
            plans.append({
                "cols": (c0, c1),
                "rs": [
                    (m0, (1 - s0) * 128, 128, k0),
                    (m1, k0 + (1 - s1) * 64, 64, k1),
                    (m2, k1 + (1 - s2) * 32, 32, k2),
                ],
                "ag": [(m2, k2, 32), (m1, k1, 64), (m0, k0, 128)],
            })

        def rs_rdma(r, s):
            msk, src_row, sz, _ = plans[r]["rs"][s]
            c0, c1 = plans[r]["cols"]
            return pltpu.make_async_remote_copy(
                src_ref=acc_ref.at[pl.ds(src_row, sz), c0:c1],
                dst_ref=rs_recv.at[pl.ds(STAGE_OFF[s], sz), c0:c1],
                send_sem=send_sems.at[r * 6 + s],
                recv_sem=recv_sems.at[r * 6 + s],
                device_id=(my ^ msk,),
                device_id_type=pl.DeviceIdType.MESH,
            )

        def ag_rdma(r, s):
            msk, row, sz = plans[r]["ag"][s]
            c0, c1 = plans[r]["cols"]
            return pltpu.make_async_remote_copy(
                src_ref=acc_ref.at[pl.ds(row, sz), c0:c1],
                dst_ref=acc_ref.at[pl.ds(row, sz), c0:c1],
                send_sem=send_sems.at[r * 6 + 3 + s],
                recv_sem=recv_sems.at[r * 6 + 3 + s],
                device_id=(my ^ msk,),
                device_id_type=pl.DeviceIdType.MESH,
            )

        for r in range(3):
            rs_rdma(r, 0).start()
        for s in range(3):
            for r in range(3):
                rs_rdma(r, s).wait()
                _, _, sz, add_row = plans[r]["rs"][s]
                c0, c1 = plans[r]["cols"]
                if s < 2:
                    nxt = rs_rdma(r, s + 1)
                else:
                    nxt = ag_rdma(r, 0)
                acc_ref[pl.ds(add_row, sz), c0:c1] += (
                    rs_recv[pl.ds(STAGE_OFF[s], sz), c0:c1])
                nxt.start()
        for s in range(3):
            for r in range(3):
                ag_rdma(r, s).wait()
                if s < 2:
                    ag_rdma(r, s + 1).start()

        out_ref[0] = acc_ref[...]

    return pl.pallas_call(
        body,
        out_shape=jax.ShapeDtypeStruct((1, SQ, D), jnp.float32),
        in_specs=[pl.BlockSpec(memory_space=pltpu.VMEM)] * 5,
        out_specs=pl.BlockSpec(memory_space=pltpu.VMEM),
        scratch_shapes=[
            pltpu.VMEM((SQ, D), jnp.float32),
            pltpu.VMEM((SQ, D), jnp.float32),
            pltpu.SemaphoreType.DMA((18,)),
            pltpu.SemaphoreType.DMA((18,)),
        ],
        compiler_params=pltpu.CompilerParams(collective_id=0),
    )(x, Wq, Wo, Wk, Wv)


# device time: 33230 ns/iter; 3.3161x vs baseline; 1.0984x over previous
import jax
import jax.numpy as jnp
from jax import lax
from jax.experimental import pallas as pl
from jax.experimental.pallas import tpu as pltpu

N_DEV = 8
SQ = 256
D = 1024
DH = 128
H_PER = 8
SCALE = 0.08838834764831843

RAIL_MASKS = ((1, 3, 4), (3, 4, 1), (4, 1, 3))
RAIL_COLS = ((0, 384), (384, 768), (768, 1024))
STAGE_OFF = (0, 128, 192)


def kernel(x, Wq, Wo, Wk, Wv):
    def body(x_ref, wq_ref, wo_ref, wk_ref, wv_ref, out_ref,
             acc_ref, rs_recv, send_sems, recv_sems):
        my = lax.axis_index("i")
        b0 = my & 1
        b1 = (my >> 1) & 1
        b2 = (my >> 2) & 1
        sels = (
            (b0 ^ b1, b1, b2),
            (b1, b2, b0),
            (b2, b0 ^ b1, b1),
        )

        barrier = pltpu.get_barrier_semaphore()
        for msk in (1, 3, 4):
            pl.semaphore_signal(barrier, inc=1, device_id=(my ^ msk,),
                                device_id_type=pl.DeviceIdType.MESH)
        pl.semaphore_wait(barrier, 3)

        plans = []
        for r, ((m0, m1, m2), (s0, s1, s2)) in enumerate(
                zip(RAIL_MASKS, sels)):
            k0 = s0 * 128
            k1 = k0 + s1 * 64
            k2 = k1 + s2 * 32
            c0, c1 = RAIL_COLS[r]
            plans.append({
                "cols": (c0, c1),
                "s0": s0,
                "rs": [
                    (m0, (1 - s0) * 128, 128, k0),
                    (m1, k0 + (1 - s1) * 64, 64, k1),
                    (m2, k1 + (1 - s2) * 32, 32, k2),
                ],
                "ag": [(m2, k2, 32), (m1, k1, 64), (m0, k0, 128)],
            })

        def rs_rdma(r, s):
            msk, src_row, sz, _ = plans[r]["rs"][s]
            c0, c1 = plans[r]["cols"]
            return pltpu.make_async_remote_copy(
                src_ref=acc_ref.at[pl.ds(src_row, sz), c0:c1],
                dst_ref=rs_recv.at[pl.ds(STAGE_OFF[s], sz), c0:c1],
                send_sem=send_sems.at[r * 6 + s],
                recv_sem=recv_sems.at[r * 6 + s],
                device_id=(my ^ msk,),
                device_id_type=pl.DeviceIdType.MESH,
            )

        def ag_rdma(r, s):
            msk, row, sz = plans[r]["ag"][s]
            c0, c1 = plans[r]["cols"]
            return pltpu.make_async_remote_copy(
                src_ref=acc_ref.at[pl.ds(row, sz), c0:c1],
                dst_ref=acc_ref.at[pl.ds(row, sz), c0:c1],
                send_sem=send_sems.at[r * 6 + 3 + s],
                recv_sem=recv_sems.at[r * 6 + 3 + s],
                device_id=(my ^ msk,),
                device_id_type=pl.DeviceIdType.MESH,
            )

        xv = x_ref[0]
        k = jnp.dot(xv, wk_ref[...], preferred_element_type=jnp.float32)
        v = jnp.dot(xv, wv_ref[...], preferred_element_type=jnp.float32)
        q = jnp.dot(xv, wq_ref[...], preferred_element_type=jnp.float32)

        def compute_rows(r0):
            outs = []
            for h in range(H_PER):
                sl = slice(h * DH, (h + 1) * DH)
                s = jnp.dot(q[r0:r0 + 128, sl], k[:, sl].T,
                            preferred_element_type=jnp.float32) * SCALE
                m = jnp.max(s, axis=-1, keepdims=True)
                p = jnp.exp(s - m)
                l = jnp.sum(p, axis=-1, keepdims=True)
                outs.append(jnp.dot(p / l, v[:, sl],
                                    preferred_element_type=jnp.float32))
            attn = jnp.concatenate(outs, axis=1)
            part = jnp.dot(attn, wo_ref[...],
                           preferred_element_type=jnp.float32)
            acc_ref[r0:r0 + 128, :] = part.astype(jnp.bfloat16)

        compute_rows(128)
        for r in range(3):
            @pl.when(plans[r]["s0"] == 0)
            def _(r=r):
                rs_rdma(r, 0).start()
        compute_rows(0)
        for r in range(3):
            @pl.when(plans[r]["s0"] == 1)
            def _(r=r):
                rs_rdma(r, 0).start()

        for s in range(3):
            for r in range(3):
                rs_rdma(r, s).wait()
                _, _, sz, add_row = plans[r]["rs"][s]
                c0, c1 = plans[r]["cols"]
                if s < 2:
                    nxt = rs_rdma(r, s + 1)
                else:
                    nxt = ag_rdma(r, 0)
                acc_ref[pl.ds(add_row, sz), c0:c1] += (
                    rs_recv[pl.ds(STAGE_OFF[s], sz), c0:c1])
                nxt.start()
        for s in range(3):
            for r in range(3):
                ag_rdma(r, s).wait()
                if s < 2:
                    ag_rdma(r, s + 1).start()

        out_ref[0] = acc_ref[...].astype(jnp.float32)

    return pl.pallas_call(
        body,
        out_shape=jax.ShapeDtypeStruct((1, SQ, D), jnp.float32),
        in_specs=[pl.BlockSpec(memory_space=pltpu.VMEM)] * 5,
        out_specs=pl.BlockSpec(memory_space=pltpu.VMEM),
        scratch_shapes=[
            pltpu.VMEM((SQ, D), jnp.bfloat16),
            pltpu.VMEM((SQ, D), jnp.bfloat16),
            pltpu.SemaphoreType.DMA((18,)),
            pltpu.SemaphoreType.DMA((18,)),
        ],
        compiler_params=pltpu.CompilerParams(collective_id=0),
    )(x, Wq, Wo, Wk, Wv)
